# baseline (speedup 1.0000x reference)
import functools

import jax
import jax.numpy as jnp
from jax import lax
from jax.experimental import pallas as pl
from jax.experimental.pallas import tpu as pltpu
from jax.experimental.pallas import tpu_sc as plsc

M = 1_000_000
N = 16384
D = 64
NS = 16
CHUNK = N // NS
W = 128
PIECES = CHUNK // W
ZROWS = 128

_MESH = plsc.VectorSubcoreMesh(
    core_axis_name="c", subcore_axis_name="s", num_cores=1
)
_CPARAMS = pltpu.CompilerParams(use_tc_tiling_on_sc=False)


def _zero_body(o_ref):
    o_ref[...] = jnp.zeros_like(o_ref)


_zeros = pl.pallas_call(
    _zero_body,
    out_shape=jax.ShapeDtypeStruct((M, D), jnp.float32),
    grid=(125,),
    out_specs=pl.BlockSpec((M // 125, D), lambda i: (i, 0)),
)


def _load_idx(idx_hbm, idx_m, s):
    pltpu.sync_copy(idx_hbm.at[pl.ds(s * PIECES, PIECES), :], idx_m)


def _k1_body(idx_hbm, aux_hbm, idx_m, pos_m):
    s = lax.axis_index("s")
    base = s * CHUNK
    _load_idx(idx_hbm, idx_m, s)

    def _mk_pos(k, _):
        j = k // (W // 16)
        c = k % (W // 16)
        pos_m[j, pl.ds(c * 16, 16)] = (
            lax.iota(jnp.int32, 16) + (base + j * W + c * 16)
        )
        return _
    lax.fori_loop(0, PIECES * (W // 16), _mk_pos, 0)

    for j in range(PIECES):
        pltpu.sync_copy(pos_m.at[j], aux_hbm.at[idx_m.at[j]])


_k1 = functools.partial(
    pl.kernel,
    out_type=jax.ShapeDtypeStruct((M,), jnp.int32),
    mesh=_MESH,
    compiler_params=_CPARAMS,
    scratch_types=[
        pltpu.VMEM((PIECES, W), jnp.int32),
        pltpu.VMEM((PIECES, W), jnp.int32),
    ],
)(_k1_body)


def _k2_body(x_hbm, idx_hbm, aux_hbm, w_hbm, sums_hbm,
             idx_m, w_m, x_v, zbuf, acc):
    s = lax.axis_index("s")
    base = s * CHUNK
    _load_idx(idx_hbm, idx_m, s)

    for j in range(PIECES):
        pltpu.sync_copy(aux_hbm.at[idx_m.at[j]], w_m.at[j])
    pltpu.sync_copy(w_m, w_hbm.at[pl.ds(s * PIECES, PIECES), :])

    def _mk_zero(k, _):
        i = k // (D // 16)
        c = k % (D // 16)
        zbuf[i, pl.ds(c * 16, 16)] = jnp.zeros((16,), jnp.float32)
        return _
    lax.fori_loop(0, ZROWS * (D // 16), _mk_zero, 0)
    for r in range(CHUNK // ZROWS):
        pltpu.sync_copy(zbuf, acc.at[pl.ds(base + r * ZROWS, ZROWS), :])

    plsc.subcore_barrier()

    for j in range(PIECES):
        pltpu.sync_copy(x_hbm.at[pl.ds(base + j * W, W), :], x_v)
        pltpu.sync_copy(x_v, acc.at[w_m.at[j]], add=True)

    plsc.subcore_barrier()

    for j in range(PIECES):
        pltpu.sync_copy(acc.at[pl.ds(base + j * W, W), :], x_v)
        pltpu.sync_copy(x_v, sums_hbm.at[pl.ds(base + j * W, W), :])


_k2 = functools.partial(
    pl.kernel,
    out_type=(
        jax.ShapeDtypeStruct((128, 128), jnp.int32),
        jax.ShapeDtypeStruct((N, D), jnp.float32),
    ),
    mesh=_MESH,
    compiler_params=_CPARAMS,
    scratch_types=[
        pltpu.VMEM((PIECES, W), jnp.int32),
        pltpu.VMEM((PIECES, W), jnp.int32),
        pltpu.VMEM((W, D), jnp.float32),
        pltpu.VMEM((ZROWS, D), jnp.float32),
        pltpu.VMEM_SHARED((N, D), jnp.float32),
    ],
)(_k2_body)


def _k3_body(idx_hbm, w_hbm, sums_hbm, out_hbm, idx_m, w_m, x_v):
    s = lax.axis_index("s")
    _load_idx(idx_hbm, idx_m, s)
    pltpu.sync_copy(w_hbm.at[pl.ds(s * PIECES, PIECES), :], w_m)

    for j in range(PIECES):
        pltpu.sync_copy(sums_hbm.at[w_m.at[j]], x_v)
        pltpu.sync_copy(x_v, out_hbm.at[idx_m.at[j]])


_k3 = functools.partial(
    pl.kernel,
    out_type=(),
    mesh=_MESH,
    compiler_params=_CPARAMS,
    scratch_types=[
        pltpu.VMEM((PIECES, W), jnp.int32),
        pltpu.VMEM((PIECES, W), jnp.int32),
        pltpu.VMEM((W, D), jnp.float32),
    ],
)(_k3_body)


def kernel(x_data, scatter_idcs, protoshape):
    idx = scatter_idcs.reshape(N).astype(jnp.int32).reshape(128, 128)
    aux = _k1(idx)
    w, sums = _k2(x_data, idx, aux)
    out_ref = jax.new_ref(_zeros())
    _k3(idx, w, sums, out_ref)
    return jax.freeze(out_ref)


# speedup vs baseline: 1.5761x; 1.5761x over previous
import functools

import jax
import jax.numpy as jnp
from jax import lax
from jax.experimental import pallas as pl
from jax.experimental.pallas import tpu as pltpu
from jax.experimental.pallas import tpu_sc as plsc

M = 1_000_000
N = 16384
D = 64
NS = 16
CHUNK = N // NS
W = 128
PIECES = CHUNK // W
ZROWS = 128

_MESH = plsc.VectorSubcoreMesh(
    core_axis_name="c", subcore_axis_name="s", num_cores=1
)
_CPARAMS = pltpu.CompilerParams(use_tc_tiling_on_sc=False)


def _zero_body(o_ref):
    o_ref[...] = jnp.zeros_like(o_ref)


_zeros = pl.pallas_call(
    _zero_body,
    out_shape=jax.ShapeDtypeStruct((M, D), jnp.float32),
    grid=(125,),
    out_specs=pl.BlockSpec((M // 125, D), lambda i: (i, 0)),
)


def _load_idx(idx_hbm, idx_m, s):
    pltpu.sync_copy(idx_hbm.at[pl.ds(s * PIECES, PIECES), :], idx_m)


def _k1_body(idx_hbm, aux_hbm, idx_m, pos_m):
    s = lax.axis_index("s")
    base = s * CHUNK
    _load_idx(idx_hbm, idx_m, s)

    def _mk_pos(k, _):
        j = k // (W // 16)
        c = k % (W // 16)
        pos_m[j, pl.ds(c * 16, 16)] = (
            lax.iota(jnp.int32, 16) + (base + j * W + c * 16)
        )
        return _
    lax.fori_loop(0, PIECES * (W // 16), _mk_pos, 0)

    for j in range(PIECES):
        pltpu.sync_copy(pos_m.at[j], aux_hbm.at[idx_m.at[j]])


_k1 = functools.partial(
    pl.kernel,
    out_type=jax.ShapeDtypeStruct((M,), jnp.int32),
    mesh=_MESH,
    compiler_params=_CPARAMS,
    scratch_types=[
        pltpu.VMEM((PIECES, W), jnp.int32),
        pltpu.VMEM((PIECES, W), jnp.int32),
    ],
)(_k1_body)


def _k2_body(x_hbm, idx_hbm, aux_hbm, w_hbm, sums_hbm,
             idx_m, w_m, x_v, zbuf, acc):
    s = lax.axis_index("s")
    base = s * CHUNK
    _load_idx(idx_hbm, idx_m, s)

    for j in range(PIECES):
        pltpu.sync_copy(aux_hbm.at[idx_m.at[j]], w_m.at[j])
    pltpu.sync_copy(w_m, w_hbm.at[pl.ds(s * PIECES, PIECES), :])

    def _mk_zero(k, _):
        i = k // (D // 16)
        c = k % (D // 16)
        zbuf[i, pl.ds(c * 16, 16)] = jnp.zeros((16,), jnp.float32)
        return _
    lax.fori_loop(0, ZROWS * (D // 16), _mk_zero, 0)
    for r in range(CHUNK // ZROWS):
        pltpu.sync_copy(zbuf, acc.at[pl.ds(base + r * ZROWS, ZROWS), :])

    plsc.subcore_barrier()

    for j in range(PIECES):
        pltpu.sync_copy(x_hbm.at[pl.ds(base + j * W, W), :], x_v)
        pltpu.sync_copy(x_v, acc.at[w_m.at[j]], add=True)

    plsc.subcore_barrier()

    for j in range(PIECES):
        pltpu.sync_copy(acc.at[pl.ds(base + j * W, W), :], x_v)
        pltpu.sync_copy(x_v, sums_hbm.at[pl.ds(base + j * W, W), :])


_k2 = functools.partial(
    pl.kernel,
    out_type=(
        jax.ShapeDtypeStruct((128, 128), jnp.int32),
        jax.ShapeDtypeStruct((N, D), jnp.float32),
    ),
    mesh=_MESH,
    compiler_params=_CPARAMS,
    scratch_types=[
        pltpu.VMEM((PIECES, W), jnp.int32),
        pltpu.VMEM((PIECES, W), jnp.int32),
        pltpu.VMEM((W, D), jnp.float32),
        pltpu.VMEM((ZROWS, D), jnp.float32),
        pltpu.VMEM_SHARED((N, D), jnp.float32),
    ],
)(_k2_body)


def _k3_body(idx_hbm, w_hbm, sums_hbm, out_hbm, idx_m, w_m, x_v):
    s = lax.axis_index("s")
    _load_idx(idx_hbm, idx_m, s)
    pltpu.sync_copy(w_hbm.at[pl.ds(s * PIECES, PIECES), :], w_m)

    for j in range(PIECES):
        pltpu.sync_copy(sums_hbm.at[w_m.at[j]], x_v)
        pltpu.sync_copy(x_v, out_hbm.at[idx_m.at[j]])


_k3 = functools.partial(
    pl.kernel,
    out_type=(),
    mesh=_MESH,
    compiler_params=_CPARAMS,
    scratch_types=[
        pltpu.VMEM((PIECES, W), jnp.int32),
        pltpu.VMEM((PIECES, W), jnp.int32),
        pltpu.VMEM((W, D), jnp.float32),
    ],
)(_k3_body)


def kernel(x_data, scatter_idcs, protoshape):
    idx = scatter_idcs.reshape(N).astype(jnp.int32).reshape(128, 128)
    aux = _k1(idx)
    w, sums = _k2(x_data, idx, aux)
    out_ref = jax.new_ref(jnp.zeros((M, D), jnp.float32))
    _k3(idx, w, sums, out_ref)
    return jax.freeze(out_ref)


# speedup vs baseline: 1.6084x; 1.0205x over previous
import functools

import jax
import jax.numpy as jnp
from jax import lax
from jax.experimental import pallas as pl
from jax.experimental.pallas import tpu as pltpu
from jax.experimental.pallas import tpu_sc as plsc

M = 1_000_000
N = 16384
D = 64
NS = 16
CHUNK = N // NS
W = 128
PIECES = CHUNK // W
ZROWS = 128

_MESH = plsc.VectorSubcoreMesh(
    core_axis_name="c", subcore_axis_name="s", num_cores=1
)
_CPARAMS = pltpu.CompilerParams(use_tc_tiling_on_sc=False)


def _zero_body(o_ref):
    o_ref[...] = jnp.zeros_like(o_ref)


_zeros = pl.pallas_call(
    _zero_body,
    out_shape=jax.ShapeDtypeStruct((M, D), jnp.float32),
    grid=(125,),
    out_specs=pl.BlockSpec((M // 125, D), lambda i: (i, 0)),
)


def _load_idx(idx_hbm, idx_m, s):
    pltpu.sync_copy(idx_hbm.at[pl.ds(s * PIECES, PIECES), :], idx_m)


def _k1_body(idx_hbm, aux_hbm, idx_m, pos_m):
    s = lax.axis_index("s")
    base = s * CHUNK
    _load_idx(idx_hbm, idx_m, s)

    def _mk_pos(k, _):
        j = k // (W // 16)
        c = k % (W // 16)
        pos_m[j, pl.ds(c * 16, 16)] = (
            lax.iota(jnp.int32, 16) + (base + j * W + c * 16)
        )
        return _
    lax.fori_loop(0, PIECES * (W // 16), _mk_pos, 0)

    for j in range(PIECES):
        pltpu.sync_copy(pos_m.at[j], aux_hbm.at[idx_m.at[j]])


_k1 = functools.partial(
    pl.kernel,
    out_type=jax.ShapeDtypeStruct((M,), jnp.int32),
    mesh=_MESH,
    compiler_params=_CPARAMS,
    scratch_types=[
        pltpu.VMEM((PIECES, W), jnp.int32),
        pltpu.VMEM((PIECES, W), jnp.int32),
    ],
)(_k1_body)


def _k2_body(x_hbm, idx_hbm, aux_hbm, out_hbm,
             idx_m, w_m, x_v, zbuf, acc):
    s = lax.axis_index("s")
    base = s * CHUNK
    _load_idx(idx_hbm, idx_m, s)

    for j in range(PIECES):
        pltpu.sync_copy(aux_hbm.at[idx_m.at[j]], w_m.at[j])

    def _mk_zero(k, _):
        i = k // (D // 16)
        c = k % (D // 16)
        zbuf[i, pl.ds(c * 16, 16)] = jnp.zeros((16,), jnp.float32)
        return _
    lax.fori_loop(0, ZROWS * (D // 16), _mk_zero, 0)
    for r in range(CHUNK // ZROWS):
        pltpu.sync_copy(zbuf, acc.at[pl.ds(base + r * ZROWS, ZROWS), :])

    plsc.subcore_barrier()

    for j in range(PIECES):
        pltpu.sync_copy(x_hbm.at[pl.ds(base + j * W, W), :], x_v)
        pltpu.sync_copy(x_v, acc.at[w_m.at[j]], add=True)

    plsc.subcore_barrier()

    for j in range(PIECES):
        pltpu.sync_copy(acc.at[w_m.at[j]], x_v)
        pltpu.sync_copy(x_v, out_hbm.at[idx_m.at[j]])


_k2 = functools.partial(
    pl.kernel,
    out_type=(),
    mesh=_MESH,
    compiler_params=_CPARAMS,
    scratch_types=[
        pltpu.VMEM((PIECES, W), jnp.int32),
        pltpu.VMEM((PIECES, W), jnp.int32),
        pltpu.VMEM((W, D), jnp.float32),
        pltpu.VMEM((ZROWS, D), jnp.float32),
        pltpu.VMEM_SHARED((N, D), jnp.float32),
    ],
)(_k2_body)


def kernel(x_data, scatter_idcs, protoshape):
    idx = scatter_idcs.reshape(N).astype(jnp.int32).reshape(128, 128)
    aux = _k1(idx)
    out_ref = jax.new_ref(jnp.zeros((M, D), jnp.float32))
    _k2(x_data, idx, aux, out_ref)
    return jax.freeze(out_ref)


# speedup vs baseline: 1.6171x; 1.0054x over previous
import functools

import jax
import jax.numpy as jnp
from jax import lax
from jax.experimental import pallas as pl
from jax.experimental.pallas import tpu as pltpu
from jax.experimental.pallas import tpu_sc as plsc

M = 1_000_000
N = 16384
D = 64
NS = 16
CHUNK = N // NS
W = 128
PIECES = CHUNK // W
ZROWS = 128

_MESH = plsc.VectorSubcoreMesh(
    core_axis_name="c", subcore_axis_name="s", num_cores=1
)
_CPARAMS = pltpu.CompilerParams(use_tc_tiling_on_sc=False)


def _load_idx(idx_hbm, idx_m, s):
    pltpu.sync_copy(idx_hbm.at[pl.ds(s * PIECES, PIECES), :], idx_m)


def _k1_body(idx_hbm, aux_hbm, idx_m, pos_m):
    s = lax.axis_index("s")
    base = s * CHUNK
    _load_idx(idx_hbm, idx_m, s)

    def _mk_pos(k, _):
        j = k // (W // 16)
        c = k % (W // 16)
        pos_m[j, pl.ds(c * 16, 16)] = (
            lax.iota(jnp.int32, 16) + (base + j * W + c * 16)
        )
        return _
    lax.fori_loop(0, PIECES * (W // 16), _mk_pos, 0)

    for j in range(PIECES):
        pltpu.sync_copy(pos_m.at[j], aux_hbm.at[idx_m.at[j]])


_k1 = functools.partial(
    pl.kernel,
    out_type=jax.ShapeDtypeStruct((M,), jnp.int32),
    mesh=_MESH,
    compiler_params=_CPARAMS,
    scratch_types=[
        pltpu.VMEM((PIECES, W), jnp.int32),
        pltpu.VMEM((PIECES, W), jnp.int32),
    ],
)(_k1_body)


def _k2_body(x_hbm, idx_hbm, aux_hbm, out_hbm,
             idx_m, w_m, x_v, zbuf, acc):
    s = lax.axis_index("s")
    base = s * CHUNK
    _load_idx(idx_hbm, idx_m, s)

    for j in range(PIECES):
        pltpu.sync_copy(aux_hbm.at[idx_m.at[j]], w_m.at[j])

    def _mk_zero(k, _):
        i = k // (D // 16)
        c = k % (D // 16)
        zbuf[i, pl.ds(c * 16, 16)] = jnp.zeros((16,), jnp.float32)
        return _
    lax.fori_loop(0, ZROWS * (D // 16), _mk_zero, 0)
    for r in range(CHUNK // ZROWS):
        pltpu.sync_copy(zbuf, acc.at[pl.ds(base + r * ZROWS, ZROWS), :])

    plsc.subcore_barrier()

    for j in range(PIECES):
        pltpu.sync_copy(x_hbm.at[pl.ds(base + j * W, W), :], x_v)
        pltpu.sync_copy(x_v, acc.at[w_m.at[j]], add=True)

    plsc.subcore_barrier()

    for j in range(PIECES):
        pltpu.sync_copy(acc.at[w_m.at[j]], x_v)
        pltpu.sync_copy(x_v, out_hbm.at[idx_m.at[j]])


_k2 = functools.partial(
    pl.kernel,
    out_type=(),
    mesh=_MESH,
    compiler_params=_CPARAMS,
    scratch_types=[
        pltpu.VMEM((PIECES, W), jnp.int32),
        pltpu.VMEM((PIECES, W), jnp.int32),
        pltpu.VMEM((W, D), jnp.float32),
        pltpu.VMEM((ZROWS, D), jnp.float32),
        pltpu.VMEM_SHARED((N, D), jnp.float32),
    ],
)(_k2_body)


def kernel(x_data, scatter_idcs, protoshape):
    idx = scatter_idcs.reshape(N).astype(jnp.int32).reshape(128, 128)
    aux = _k1(idx)
    out_ref = jax.new_ref(jnp.zeros((M, D), jnp.float32))
    _k2(x_data, idx, aux, out_ref)
    return jax.freeze(out_ref)
